# baseline (speedup 1.0000x reference)
import functools

import jax
import jax.numpy as jnp
from jax import lax
from jax.experimental import pallas as pl
from jax.experimental.pallas import tpu as pltpu
from jax.experimental.pallas import tpu_sc as plsc

D = 32
GROUP = 4
WIDE = GROUP * D
NUM_CORES = 2
NUM_SUBCORES = 16
NW = NUM_CORES * NUM_SUBCORES
CHUNK = 256
NBUF = 2


def _make_embed(batch: int, fields: int, v_groups: int):
    b_per_w = batch // NW
    cpf = b_per_w // CHUNK
    n_chunks = fields * cpf
    mesh = plsc.VectorSubcoreMesh(core_axis_name="c", subcore_axis_name="s")

    @functools.partial(
        pl.kernel,
        mesh=mesh,
        out_type=jax.ShapeDtypeStruct((fields, D, batch), jnp.float32),
        scratch_types=[
            pltpu.VMEM((CHUNK,), jnp.int32),
            pltpu.VMEM((CHUNK,), jnp.int32),
            pltpu.VMEM((CHUNK,), jnp.int32),
            pltpu.VMEM((CHUNK,), jnp.int32),
            pltpu.VMEM((CHUNK, WIDE), jnp.float32),
            pltpu.VMEM((CHUNK, WIDE), jnp.float32),
            pltpu.VMEM((D, CHUNK), jnp.float32),
            pltpu.VMEM((D, CHUNK), jnp.float32),
            pltpu.SemaphoreType.DMA,
            pltpu.SemaphoreType.DMA,
            pltpu.SemaphoreType.DMA,
            pltpu.SemaphoreType.DMA,
            pltpu.SemaphoreType.DMA,
        ],
        compiler_params=pltpu.CompilerParams(
            use_tc_tiling_on_sc=True, needs_layout_passes=False
        ),
    )
    def embed(
        table_hbm, idx_hbm, out_hbm,
        ix0, ix1, ig0, ig1, wd0, wd1, rt0, rt1,
        is0, is1, gs, os0, os1,
    ):
        wid = lax.axis_index("s") * NUM_CORES + lax.axis_index("c")
        b0 = wid * b_per_w
        idx_v = (ix0, ix1)
        idxg_v = (ig0, ig1)
        wide_v = (wd0, wd1)
        rowst_v = (rt0, rt1)
        isem = (is0, is1)
        osem = (os0, os1)

        def flat_off(i):
            f, c2 = divmod(i, cpf)
            return f * batch + b0 + c2 * CHUNK

        def start_icopy(i):
            return pltpu.async_copy(
                idx_hbm.at[pl.ds(flat_off(i), CHUNK)],
                idx_v[i % NBUF],
                isem[i % NBUF],
            )

        def idxg_pass(b):
            def body(k, carry):
                v = idx_v[b][pl.ds(k * 16, 16)]
                idxg_v[b][pl.ds(k * 16, 16)] = lax.shift_right_logical(v, 2)
                return carry

            lax.fori_loop(0, CHUNK // 16, body, 0)

        def start_gather(b):
            return pltpu.async_copy(table_hbm.at[idxg_v[b]], wide_v[b], gs)

        def selection(b):
            def body(k, carry):
                cvec = lax.iota(jnp.int32, 16) + k * 16
                idxvec = idx_v[b][pl.ds(k * 16, 16)]
                svec = (idxvec & 3) * D

                def jbody(j, carry2):
                    jvec = lax.broadcast(j, (16,))
                    v = plsc.load_gather(wide_v[b], [cvec, svec + jvec])
                    plsc.store_scatter(rowst_v[b], [jvec, cvec], v)
                    return carry2

                lax.fori_loop(0, D, jbody, 0)
                return carry

            lax.fori_loop(0, CHUNK // 16, body, 0)

        def start_ocopy(b, i):
            f, c2 = divmod(i, cpf)
            return pltpu.async_copy(
                rowst_v[b],
                out_hbm.at[f, :, pl.ds(b0 + c2 * CHUNK, CHUNK)],
                osem[b],
            )

        icopy = [start_icopy(0), start_icopy(1)]
        icopy[0].wait()
        idxg_pass(0)
        gcopy = start_gather(0)

        ocopy = [None] * NBUF
        for i in range(n_chunks):
            b = i % NBUF
            bn = (i + 1) % NBUF
            if i + 1 < n_chunks:
                icopy[bn].wait()
                idxg_pass(bn)
            gcopy.wait()
            if i + 1 < n_chunks:
                gcopy = start_gather(bn)
            if ocopy[b] is not None:
                ocopy[b].wait()
            selection(b)
            ocopy[b] = start_ocopy(b, i)
            if i + NBUF < n_chunks:
                icopy[b] = start_icopy(i + NBUF)
        for b in range(NBUF):
            if ocopy[b] is not None:
                ocopy[b].wait()

    return embed


def kernel(n_flat, embedding):
    batch, fields = n_flat.shape
    vocab = embedding.shape[0]
    idx_fm = n_flat.T.reshape(-1).astype(jnp.int32)
    table_wide = embedding.reshape(vocab // GROUP, WIDE)
    out_fjb = _make_embed(batch, fields, vocab // GROUP)(table_wide, idx_fm)
    return out_fjb.transpose(2, 0, 1)


# speedup vs baseline: 1.0380x; 1.0380x over previous
import functools

import jax
import jax.numpy as jnp
from jax import lax
from jax.experimental import pallas as pl
from jax.experimental.pallas import tpu as pltpu
from jax.experimental.pallas import tpu_sc as plsc

D = 32
NUM_CORES = 2
NUM_SUBCORES = 16
NW = NUM_CORES * NUM_SUBCORES
CHUNK = 512
NBUF = 2


def _make_embed(batch: int, fields: int):
    b_per_w = batch // NW
    n_chunks = fields
    mesh = plsc.VectorSubcoreMesh(core_axis_name="c", subcore_axis_name="s")

    @functools.partial(
        pl.kernel,
        mesh=mesh,
        out_type=jax.ShapeDtypeStruct((fields * D, batch), jnp.float32),
        scratch_types=[
            pltpu.VMEM((CHUNK,), jnp.int32),
            pltpu.VMEM((CHUNK,), jnp.int32),
            pltpu.VMEM((CHUNK, D), jnp.float32),
            pltpu.VMEM((CHUNK, D), jnp.float32),
            pltpu.VMEM((D, CHUNK), jnp.float32),
            pltpu.VMEM((D, CHUNK), jnp.float32),
            pltpu.SemaphoreType.DMA,
            pltpu.SemaphoreType.DMA,
            pltpu.SemaphoreType.DMA,
            pltpu.SemaphoreType.DMA,
            pltpu.SemaphoreType.DMA,
        ],
        compiler_params=pltpu.CompilerParams(
            use_tc_tiling_on_sc=False, needs_layout_passes=False
        ),
    )
    def embed(
        table_hbm, idx_hbm, out_hbm,
        ix0, ix1, rw0, rw1, rt0, rt1,
        is0, is1, gs, os0, os1,
    ):
        wid = lax.axis_index("s") * NUM_CORES + lax.axis_index("c")
        b0 = wid * b_per_w
        idx_v = (ix0, ix1)
        rows_v = (rw0, rw1)
        rowst_v = (rt0, rt1)
        isem = (is0, is1)
        osem = (os0, os1)

        def start_icopy(f):
            return pltpu.async_copy(
                idx_hbm.at[pl.ds(f * batch + b0, CHUNK)],
                idx_v[f % NBUF],
                isem[f % NBUF],
            )

        def start_gather(b):
            return pltpu.async_copy(table_hbm.at[idx_v[b]], rows_v[b], gs)

        def transpose(b):
            iota = lax.iota(jnp.int32, 16)
            iota2 = iota + 16

            def body(c, carry):
                cvec = lax.broadcast(c, (16,))
                v0 = rows_v[b][c, pl.ds(0, 16)]
                v1 = rows_v[b][c, pl.ds(16, 16)]
                plsc.store_scatter(rowst_v[b], [iota, cvec], v0)
                plsc.store_scatter(rowst_v[b], [iota2, cvec], v1)
                return carry

            lax.fori_loop(0, CHUNK, body, 0)

        def start_ocopy(b, f):
            return pltpu.async_copy(
                rowst_v[b],
                out_hbm.at[pl.ds(f * D, D), pl.ds(b0, CHUNK)],
                osem[b],
            )

        icopy = [start_icopy(0), start_icopy(1)]
        icopy[0].wait()
        gcopy = start_gather(0)

        ocopy = [None] * NBUF
        for f in range(n_chunks):
            b = f % NBUF
            bn = (f + 1) % NBUF
            gcopy.wait()
            if f + 1 < n_chunks:
                icopy[bn].wait()
                gcopy = start_gather(bn)
            if ocopy[b] is not None:
                ocopy[b].wait()
            transpose(b)
            ocopy[b] = start_ocopy(b, f)
            if f + NBUF < n_chunks:
                icopy[b] = start_icopy(f + NBUF)
        for b in range(NBUF):
            if ocopy[b] is not None:
                ocopy[b].wait()

    return embed


def kernel(n_flat, embedding):
    batch, fields = n_flat.shape
    idx_fm = n_flat.T.reshape(-1).astype(jnp.int32)
    out2d = _make_embed(batch, fields)(embedding, idx_fm)
    return out2d.reshape(fields, D, batch).transpose(2, 0, 1)


# speedup vs baseline: 1.1698x; 1.1270x over previous
import functools

import jax
import jax.numpy as jnp
from jax import lax
from jax.experimental import pallas as pl
from jax.experimental.pallas import tpu as pltpu
from jax.experimental.pallas import tpu_sc as plsc

D = 32
NUM_CORES = 2
NUM_SUBCORES = 16
NW = NUM_CORES * NUM_SUBCORES
CHUNK = 1664
NBUF = 2


def _make_embed(b_total: int):
    b_per_w = b_total // NW
    n_chunks = b_per_w // CHUNK
    mesh = plsc.VectorSubcoreMesh(core_axis_name="c", subcore_axis_name="s")

    @functools.partial(
        pl.kernel,
        mesh=mesh,
        out_type=jax.ShapeDtypeStruct((b_total, D), jnp.float32),
        scratch_types=[
            pltpu.VMEM((CHUNK,), jnp.int32),
            pltpu.VMEM((CHUNK,), jnp.int32),
            pltpu.VMEM((CHUNK, D), jnp.float32),
            pltpu.VMEM((CHUNK, D), jnp.float32),
            pltpu.SemaphoreType.DMA,
            pltpu.SemaphoreType.DMA,
            pltpu.SemaphoreType.DMA,
            pltpu.SemaphoreType.DMA,
            pltpu.SemaphoreType.DMA,
        ],
        compiler_params=pltpu.CompilerParams(use_tc_tiling_on_sc=False),
    )
    def embed(
        table_hbm, idx_hbm, out_hbm, ix0, ix1, rw0, rw1, is0, is1, gs, os0, os1
    ):
        wid = lax.axis_index("s") * NUM_CORES + lax.axis_index("c")
        base = wid * b_per_w
        idx_v = (ix0, ix1)
        rows_v = (rw0, rw1)
        isem = (is0, is1)
        osem = (os0, os1)

        icopy = [
            pltpu.async_copy(
                idx_hbm.at[pl.ds(base + b * CHUNK, CHUNK)], idx_v[b], isem[b]
            )
            for b in range(NBUF)
        ]
        ocopy = [None] * NBUF
        for i in range(n_chunks):
            b = i % NBUF
            icopy[b].wait()
            if ocopy[b] is not None:
                ocopy[b].wait()
            pltpu.async_copy(table_hbm.at[idx_v[b]], rows_v[b], gs).wait()
            if i + NBUF < n_chunks:
                icopy[b] = pltpu.async_copy(
                    idx_hbm.at[pl.ds(base + (i + NBUF) * CHUNK, CHUNK)],
                    idx_v[b],
                    isem[b],
                )
            ocopy[b] = pltpu.async_copy(
                rows_v[b], out_hbm.at[pl.ds(base + i * CHUNK, CHUNK)], osem[b]
            )
        for b in range(NBUF):
            if ocopy[b] is not None:
                ocopy[b].wait()

    return embed


def kernel(n_flat, embedding):
    batch, fields = n_flat.shape
    idx_fm = n_flat.T.reshape(-1).astype(jnp.int32)
    out2 = _make_embed(batch * fields)(embedding, idx_fm)
    return out2.reshape(fields, batch, D).transpose(1, 0, 2)
